# baseline (speedup 1.0000x reference)
import functools
import math

import jax
import jax.numpy as jnp
import numpy as np
from jax import lax
from jax.experimental import pallas as pl
from jax.experimental.pallas import tpu as pltpu
from jax.experimental.pallas import tpu_sc as plsc

R_MIN = 0.0
R_MAX = 10.0
R_BINS = 20
Z_MIN = -2.0
Z_MAX = 2.0
Z_BINS = 20
DR = (R_MAX - R_MIN) / R_BINS
DZ = (Z_MAX - Z_MIN) / Z_BINS
N = 10000000

NW = 32
C = 8192
NFULL = N // C
COMMON = (NFULL // NW) * NW
EXTRA_W = NFULL - COMMON
TAIL = N - NFULL * C
TAIL_W = EXTRA_W % NW
TPB = 402
TPAD = 416


def _build_tables():
    t = np.zeros(21, np.float64)
    for k in range(1, 21):
        c = np.float32(k) / np.float32(2.0)
        tk = np.float32(c) * np.float32(c)
        while True:
            dn = np.nextafter(tk, np.float32(0.0), dtype=np.float32)
            if np.float32(np.sqrt(dn)) >= c:
                tk = dn
            else:
                break
        t[k] = np.float64(tk)
    cand = np.zeros(TPAD, np.int32)
    thr = np.full(TPAD, np.inf, np.float32)
    for b in range(TPB):
        lo = b / 4.0
        i_lo = sum(1 for k in range(1, 20) if t[k] <= lo)
        cand[b] = i_lo
        if i_lo + 1 <= 19:
            thr[b] = np.float32(t[i_lo + 1])
    return cand, thr, np.float32(t[20])


_CAND_TAB, _THR_TAB, _T20 = _build_tables()

_redge = (np.arange(R_BINS + 1, dtype=np.float32) * np.float32(DR)).astype(np.float32)
_area = np.float32(math.pi) * (_redge[1:] * _redge[1:] - _redge[:-1] * _redge[:-1])
_vol = (_area * np.float32(DZ)).astype(np.float32)
_VOL_FLAT = np.repeat(_vol, Z_BINS).reshape(1, R_BINS * Z_BINS)


def _sc_body(pos_hbm, mass_hbm, cand_hbm, thr_hbm, out_hbm,
             pv0, pv1, mv0, mv1, cand_v, thr_v, hist, hf,
             sem_p0, sem_p1, sem_m0, sem_m1):
    cid = lax.axis_index("c")
    sid = lax.axis_index("s")
    wid = cid * 16 + sid

    lane = jnp.arange(16, dtype=jnp.int32)
    i3 = lane * 3
    zero16 = jnp.zeros((16,), jnp.float32)

    pltpu.sync_copy(cand_hbm, cand_v)
    pltpu.sync_copy(thr_hbm, thr_v)

    def _zero(a, _):
        hist[a] = zero16
        return 0
    lax.fori_loop(0, R_BINS * Z_BINS, _zero, 0)

    def chunk_base(t):
        return (wid + NW * t) * C

    def start(t, bufs, sems, n=C):
        b = chunk_base(t)
        pv, mv = bufs
        sp, sm = sems
        pltpu.async_copy(pos_hbm.at[pl.ds(b * 3, n * 3)], pv.at[pl.ds(0, n * 3)], sp)
        pltpu.async_copy(mass_hbm.at[pl.ds(b, n)], mv.at[pl.ds(0, n)], sm)

    def wait(t, bufs, sems, n=C):
        b = chunk_base(t)
        pv, mv = bufs
        sp, sm = sems
        pltpu.make_async_copy(pos_hbm.at[pl.ds(b * 3, n * 3)], pv.at[pl.ds(0, n * 3)], sp).wait()
        pltpu.make_async_copy(mass_hbm.at[pl.ds(b, n)], mv.at[pl.ds(0, n)], sm).wait()

    def process(bufs, ngroups):
        pv, mv = bufs

        def body(g2, _):
            for u in range(2):
                k = g2 * 2 + u
                sl = pl.ds(k * 16, 16)
                ix = k * 48 + i3
                x = plsc.load_gather(pv, [ix])
                y = plsc.load_gather(pv, [ix + 1])
                z = plsc.load_gather(pv, [ix + 2])
                m = mv[sl]
                r2 = x * x + y * y
                s = jnp.minimum(r2 * 4.0, 401.0)
                b = s.astype(jnp.int32)
                cnd = plsc.load_gather(cand_v, [b])
                thr = plsc.load_gather(thr_v, [b])
                i = cnd + jnp.where(r2 >= thr, 1, 0).astype(jnp.int32)
                q = (z - Z_MIN) / jnp.float32(DZ)
                jj = jnp.minimum(q.astype(jnp.int32), Z_BINS - 1)
                mask = (q >= 0.0) & (q < float(Z_BINS)) & (r2 < _T20)
                flat = i * Z_BINS + jj
                plsc.addupdate_scatter(hist, [flat, lane], m, mask=mask)
            return 0
        lax.fori_loop(0, ngroups // 2, body, 0)

    bufs0 = (pv0, mv0)
    bufs1 = (pv1, mv1)
    sems0 = (sem_p0, sem_m0)
    sems1 = (sem_p1, sem_m1)

    nring = COMMON // NW
    start(0, bufs0, sems0)

    def ring(it, _):
        t0 = it * 2
        start(t0 + 1, bufs1, sems1)
        wait(t0, bufs0, sems0)
        process(bufs0, C // 16)

        @pl.when(t0 + 2 < nring)
        def _():
            start(t0 + 2, bufs0, sems0)
        wait(t0 + 1, bufs1, sems1)
        process(bufs1, C // 16)
        return 0
    lax.fori_loop(0, nring // 2, ring, 0)

    @pl.when(wid < EXTRA_W)
    def _():
        start(nring, bufs0, sems0)
        wait(nring, bufs0, sems0)
        process(bufs0, C // 16)

    @pl.when(wid == TAIL_W)
    def _():
        t_tail = (NFULL - TAIL_W) // NW
        start(t_tail, bufs1, sems1, n=TAIL)
        wait(t_tail, bufs1, sems1, n=TAIL)
        process(bufs1, TAIL // 16)

    for c in range(25):
        binc = c * 16 + lane
        acc = zero16
        for l in range(16):
            acc = acc + plsc.load_gather(hist, [binc, jnp.full((16,), l, jnp.int32)])
        hf[pl.ds(c * 16, 16)] = acc

    pltpu.sync_copy(hf, out_hbm.at[wid])


@functools.cache
def _make_sc_hist():
    return pl.kernel(
        _sc_body,
        out_type=jax.ShapeDtypeStruct((NW, R_BINS * Z_BINS), jnp.float32),
        mesh=plsc.VectorSubcoreMesh(
            core_axis_name="c", subcore_axis_name="s",
            num_cores=2, num_subcores=16),
        compiler_params=pltpu.CompilerParams(needs_layout_passes=False),
        scratch_types=(
            [pltpu.VMEM((C * 3,), jnp.float32),
             pltpu.VMEM((C * 3,), jnp.float32),
             pltpu.VMEM((C,), jnp.float32),
             pltpu.VMEM((C,), jnp.float32),
             pltpu.VMEM((TPAD,), jnp.int32),
             pltpu.VMEM((TPAD,), jnp.float32),
             pltpu.VMEM((R_BINS * Z_BINS, 16), jnp.float32),
             pltpu.VMEM((R_BINS * Z_BINS,), jnp.float32)]
            + [pltpu.SemaphoreType.DMA for _ in range(4)]
        ),
    )


def _tc_reduce_body(p_ref, v_ref, o_ref):
    o_ref[...] = jnp.sum(p_ref[...], axis=0, keepdims=True) / v_ref[...]


_tc_reduce = pl.pallas_call(
    _tc_reduce_body,
    out_shape=jax.ShapeDtypeStruct((1, R_BINS * Z_BINS), jnp.float32),
)


def kernel(positions, masses):
    pflat = jax.lax.optimization_barrier(positions.reshape(-1))
    partials = _make_sc_hist()(pflat, masses,
                               jnp.asarray(_CAND_TAB), jnp.asarray(_THR_TAB))
    dens = _tc_reduce(partials, jnp.asarray(_VOL_FLAT))
    return dens.reshape(R_BINS, Z_BINS)


# speedup vs baseline: 11.4720x; 11.4720x over previous
import functools
import math

import jax
import jax.numpy as jnp
import numpy as np
from jax import lax
from jax.experimental import pallas as pl
from jax.experimental.pallas import tpu as pltpu
from jax.experimental.pallas import tpu_sc as plsc

R_MIN = 0.0
R_MAX = 10.0
R_BINS = 20
Z_MIN = -2.0
Z_MAX = 2.0
Z_BINS = 20
DR = (R_MAX - R_MIN) / R_BINS
DZ = (Z_MAX - Z_MIN) / Z_BINS
N = 10000000

NW = 32
C = 8192
NFULL = N // C
COMMON = (NFULL // NW) * NW
EXTRA_W = NFULL - COMMON
TAIL = N - NFULL * C
TAIL_W = EXTRA_W % NW
TPB = 402
TPAD = 416


def _build_tables():
    t = np.zeros(21, np.float64)
    for k in range(1, 21):
        c = np.float32(k) / np.float32(2.0)
        tk = np.float32(c) * np.float32(c)
        while True:
            dn = np.nextafter(tk, np.float32(0.0), dtype=np.float32)
            if np.float32(np.sqrt(dn)) >= c:
                tk = dn
            else:
                break
        t[k] = np.float64(tk)
    cand = np.zeros(TPAD, np.int32)
    thr = np.full(TPAD, np.inf, np.float32)
    for b in range(TPB):
        lo = b / 4.0
        i_lo = sum(1 for k in range(1, 20) if t[k] <= lo)
        cand[b] = i_lo
        if i_lo + 1 <= 19:
            thr[b] = np.float32(t[i_lo + 1])
    return cand, thr, np.float32(t[20])


_CAND_TAB, _THR_TAB, _T20 = _build_tables()

_redge = (np.arange(R_BINS + 1, dtype=np.float32) * np.float32(DR)).astype(np.float32)
_area = np.float32(math.pi) * (_redge[1:] * _redge[1:] - _redge[:-1] * _redge[:-1])
_vol = (_area * np.float32(DZ)).astype(np.float32)
_VOL_FLAT = np.repeat(_vol, Z_BINS).reshape(1, R_BINS * Z_BINS)


def _sc_body(pos_hbm, mass_hbm, cand_hbm, thr_hbm, out_hbm,
             pv0, pv1, mv0, mv1, cand_v, thr_v, hist, hf,
             sem_p0, sem_p1, sem_m0, sem_m1):
    cid = lax.axis_index("c")
    sid = lax.axis_index("s")
    wid = cid * 16 + sid

    lane = jnp.arange(16, dtype=jnp.int32)
    i3 = lane * 3
    zero16 = jnp.zeros((16,), jnp.float32)

    pltpu.sync_copy(cand_hbm, cand_v)
    pltpu.sync_copy(thr_hbm, thr_v)

    def _zero(a, _):
        hist[a] = zero16
        return 0
    lax.fori_loop(0, R_BINS * Z_BINS, _zero, 0)

    def chunk_base(t):
        return (wid + NW * t) * C

    def start(t, bufs, sems, n=C):
        b = chunk_base(t)
        pv, mv = bufs
        sp, sm = sems
        pltpu.async_copy(pos_hbm.at[:, pl.ds(b, n)], pv.at[:, pl.ds(0, n)], sp)
        pltpu.async_copy(mass_hbm.at[pl.ds(b, n)], mv.at[pl.ds(0, n)], sm)

    def wait(t, bufs, sems, n=C):
        b = chunk_base(t)
        pv, mv = bufs
        sp, sm = sems
        pltpu.make_async_copy(pos_hbm.at[:, pl.ds(b, n)], pv.at[:, pl.ds(0, n)], sp).wait()
        pltpu.make_async_copy(mass_hbm.at[pl.ds(b, n)], mv.at[pl.ds(0, n)], sm).wait()

    def process(bufs, ngroups):
        pv, mv = bufs

        def body(g2, _):
            for u in range(2):
                k = g2 * 2 + u
                sl = pl.ds(k * 16, 16)
                x = pv[0, sl]
                y = pv[1, sl]
                z = pv[2, sl]
                m = mv[sl]
                r2 = x * x + y * y
                s = jnp.minimum(r2 * 4.0, 401.0)
                b = s.astype(jnp.int32)
                cnd = plsc.load_gather(cand_v, [b])
                thr = plsc.load_gather(thr_v, [b])
                i = cnd + jnp.where(r2 >= thr, 1, 0).astype(jnp.int32)
                q = (z - Z_MIN) / jnp.float32(DZ)
                jj = jnp.minimum(q.astype(jnp.int32), Z_BINS - 1)
                mask = (q >= 0.0) & (q < float(Z_BINS)) & (r2 < _T20)
                flat = i * Z_BINS + jj
                plsc.addupdate_scatter(hist, [flat, lane], m, mask=mask)
            return 0
        lax.fori_loop(0, ngroups // 2, body, 0)

    bufs0 = (pv0, mv0)
    bufs1 = (pv1, mv1)
    sems0 = (sem_p0, sem_m0)
    sems1 = (sem_p1, sem_m1)

    nring = COMMON // NW
    start(0, bufs0, sems0)

    def ring(it, _):
        t0 = it * 2
        start(t0 + 1, bufs1, sems1)
        wait(t0, bufs0, sems0)
        process(bufs0, C // 16)

        @pl.when(t0 + 2 < nring)
        def _():
            start(t0 + 2, bufs0, sems0)
        wait(t0 + 1, bufs1, sems1)
        process(bufs1, C // 16)
        return 0
    lax.fori_loop(0, nring // 2, ring, 0)

    @pl.when(wid < EXTRA_W)
    def _():
        start(nring, bufs0, sems0)
        wait(nring, bufs0, sems0)
        process(bufs0, C // 16)

    @pl.when(wid == TAIL_W)
    def _():
        t_tail = (NFULL - TAIL_W) // NW
        start(t_tail, bufs1, sems1, n=TAIL)
        wait(t_tail, bufs1, sems1, n=TAIL)
        process(bufs1, TAIL // 16)

    for c in range(25):
        binc = c * 16 + lane
        acc = zero16
        for l in range(16):
            acc = acc + plsc.load_gather(hist, [binc, jnp.full((16,), l, jnp.int32)])
        hf[pl.ds(c * 16, 16)] = acc

    pltpu.sync_copy(hf, out_hbm.at[wid])


@functools.cache
def _make_sc_hist():
    return pl.kernel(
        _sc_body,
        out_type=jax.ShapeDtypeStruct((NW, R_BINS * Z_BINS), jnp.float32),
        mesh=plsc.VectorSubcoreMesh(
            core_axis_name="c", subcore_axis_name="s",
            num_cores=2, num_subcores=16),
        compiler_params=pltpu.CompilerParams(
            needs_layout_passes=False, use_tc_tiling_on_sc=False),
        scratch_types=(
            [pltpu.VMEM((3, C), jnp.float32),
             pltpu.VMEM((3, C), jnp.float32),
             pltpu.VMEM((C,), jnp.float32),
             pltpu.VMEM((C,), jnp.float32),
             pltpu.VMEM((TPAD,), jnp.int32),
             pltpu.VMEM((TPAD,), jnp.float32),
             pltpu.VMEM((R_BINS * Z_BINS, 16), jnp.float32),
             pltpu.VMEM((R_BINS * Z_BINS,), jnp.float32)]
            + [pltpu.SemaphoreType.DMA for _ in range(4)]
        ),
    )


def _tc_reduce_body(p_ref, v_ref, o_ref):
    o_ref[...] = jnp.sum(p_ref[...], axis=0, keepdims=True) / v_ref[...]


_tc_reduce = pl.pallas_call(
    _tc_reduce_body,
    out_shape=jax.ShapeDtypeStruct((1, R_BINS * Z_BINS), jnp.float32),
)


def kernel(positions, masses):
    pos_t = jnp.einsum("ij,jk->ki", positions, jnp.eye(3, dtype=jnp.float32),
                       precision=lax.Precision.HIGHEST)
    partials = _make_sc_hist()(pos_t, masses,
                               jnp.asarray(_CAND_TAB), jnp.asarray(_THR_TAB))
    dens = _tc_reduce(partials, jnp.asarray(_VOL_FLAT))
    return dens.reshape(R_BINS, Z_BINS)


# speedup vs baseline: 42.9276x; 3.7419x over previous
import functools
import math

import jax
import jax.numpy as jnp
import numpy as np
from jax import lax
from jax.experimental import pallas as pl
from jax.experimental.pallas import tpu as pltpu
from jax.experimental.pallas import tpu_sc as plsc

R_MIN = 0.0
R_MAX = 10.0
R_BINS = 20
Z_MIN = -2.0
Z_MAX = 2.0
Z_BINS = 20
DR = (R_MAX - R_MIN) / R_BINS
DZ = (Z_MAX - Z_MIN) / Z_BINS
N = 10000000

NW = 32
NSEG = 4
TPB = 402
TPAD = 416


def _build_tables():
    t = np.zeros(21, np.float64)
    for k in range(1, 21):
        c = np.float32(k) / np.float32(2.0)
        tk = np.float32(c) * np.float32(c)
        while True:
            dn = np.nextafter(tk, np.float32(0.0), dtype=np.float32)
            if np.float32(np.sqrt(dn)) >= c:
                tk = dn
            else:
                break
        t[k] = np.float64(tk)
    cand = np.zeros(TPAD, np.int32)
    thr = np.full(TPAD, np.inf, np.float32)
    for b in range(TPB):
        lo = b / 4.0
        i_lo = sum(1 for k in range(1, 20) if t[k] <= lo)
        cand[b] = i_lo
        if i_lo + 1 <= 19:
            thr[b] = np.float32(t[i_lo + 1])
    return cand, thr, np.float32(t[20])


_CAND_TAB, _THR_TAB, _T20 = _build_tables()

_redge = (np.arange(R_BINS + 1, dtype=np.float32) * np.float32(DR)).astype(np.float32)
_area = np.float32(math.pi) * (_redge[1:] * _redge[1:] - _redge[:-1] * _redge[:-1])
_vol = (_area * np.float32(DZ)).astype(np.float32)
_VOL_FLAT = np.repeat(_vol, Z_BINS).reshape(1, R_BINS * Z_BINS)


def _make_sc_body(L, C):
    nfull = L // C
    nring = ((nfull // NW) // 2) * 2
    extra = nfull - nring * NW
    tail = L - nfull * C
    tail_w = extra % NW
    assert extra <= NW and tail % 16 == 0 and (tail // 16) % 2 == 0
    assert C % 8 == 0 and (tail == 0 or tail % 8 == 0)

    def body(xs_hbm, ys_hbm, zs_hbm, mass_hbm, cand_hbm, thr_hbm, out_hbm,
             xv0, xv1, yv0, yv1, zv0, zv1, mv0, mv1, cand_v, thr_v, hist, hf,
             sem_x0, sem_x1, sem_y0, sem_y1, sem_z0, sem_z1, sem_m0, sem_m1):
        cid = lax.axis_index("c")
        sid = lax.axis_index("s")
        wid = cid * 16 + sid

        lane = jnp.arange(16, dtype=jnp.int32)
        zero16 = jnp.zeros((16,), jnp.float32)

        pltpu.sync_copy(cand_hbm, cand_v)
        pltpu.sync_copy(thr_hbm, thr_v)

        def _zero(a, _):
            hist[a] = zero16
            return 0
        lax.fori_loop(0, R_BINS * Z_BINS, _zero, 0)

        def chunk_base(t):
            return (wid + NW * t) * C

        def start(t, bufs, sems, n=C):
            b = chunk_base(t)
            for src, dst, sem in zip((xs_hbm, ys_hbm, zs_hbm, mass_hbm), bufs, sems):
                pltpu.async_copy(src.at[pl.ds(b, n)], dst.at[pl.ds(0, n)], sem)

        def wait(t, bufs, sems, n=C):
            b = chunk_base(t)
            for src, dst, sem in zip((xs_hbm, ys_hbm, zs_hbm, mass_hbm), bufs, sems):
                pltpu.make_async_copy(src.at[pl.ds(b, n)], dst.at[pl.ds(0, n)], sem).wait()

        def process(bufs, ngroups):
            xv, yv, zv, mv = bufs

            def pbody(g2, _):
                for u in range(2):
                    k = g2 * 2 + u
                    sl = pl.ds(k * 16, 16)
                    x = xv[sl]
                    y = yv[sl]
                    z = zv[sl]
                    m = mv[sl]
                    r2 = x * x + y * y
                    s = jnp.minimum(r2 * 4.0, 401.0)
                    b = s.astype(jnp.int32)
                    cnd = plsc.load_gather(cand_v, [b])
                    thr = plsc.load_gather(thr_v, [b])
                    i = cnd + jnp.where(r2 >= thr, 1, 0).astype(jnp.int32)
                    q = (z - Z_MIN) / jnp.float32(DZ)
                    jj = jnp.minimum(q.astype(jnp.int32), Z_BINS - 1)
                    mask = (q >= 0.0) & (q < float(Z_BINS)) & (r2 < _T20)
                    flat = i * Z_BINS + jj
                    plsc.addupdate_scatter(hist, [flat, lane], m, mask=mask)
                return 0
            lax.fori_loop(0, ngroups // 2, pbody, 0)

        bufs0 = (xv0, yv0, zv0, mv0)
        bufs1 = (xv1, yv1, zv1, mv1)
        sems0 = (sem_x0, sem_y0, sem_z0, sem_m0)
        sems1 = (sem_x1, sem_y1, sem_z1, sem_m1)

        start(0, bufs0, sems0)

        def ring(it, _):
            t0 = it * 2
            start(t0 + 1, bufs1, sems1)
            wait(t0, bufs0, sems0)
            process(bufs0, C // 16)

            @pl.when(t0 + 2 < nring)
            def _():
                start(t0 + 2, bufs0, sems0)
            wait(t0 + 1, bufs1, sems1)
            process(bufs1, C // 16)
            return 0
        lax.fori_loop(0, nring // 2, ring, 0)

        if extra:
            @pl.when(wid < extra)
            def _():
                start(nring, bufs0, sems0)
                wait(nring, bufs0, sems0)
                process(bufs0, C // 16)

        if tail:
            @pl.when(wid == tail_w)
            def _():
                t_tail = (nfull - tail_w) // NW
                start(t_tail, bufs1, sems1, n=tail)
                wait(t_tail, bufs1, sems1, n=tail)
                process(bufs1, tail // 16)

        for c in range(25):
            binc = c * 16 + lane
            acc = zero16
            for l in range(16):
                acc = acc + plsc.load_gather(hist, [binc, jnp.full((16,), l, jnp.int32)])
            hf[pl.ds(c * 16, 16)] = acc

        pltpu.sync_copy(hf, out_hbm.at[wid])

    return body


@functools.cache
def _make_sc_hist(L, C):
    return pl.kernel(
        _make_sc_body(L, C),
        out_type=jax.ShapeDtypeStruct((NW, R_BINS * Z_BINS), jnp.float32),
        mesh=plsc.VectorSubcoreMesh(
            core_axis_name="c", subcore_axis_name="s",
            num_cores=2, num_subcores=16),
        compiler_params=pltpu.CompilerParams(needs_layout_passes=False),
        scratch_types=(
            [pltpu.VMEM((C,), jnp.float32) for _ in range(8)]
            + [pltpu.VMEM((TPAD,), jnp.int32),
               pltpu.VMEM((TPAD,), jnp.float32),
               pltpu.VMEM((R_BINS * Z_BINS, 16), jnp.float32),
               pltpu.VMEM((R_BINS * Z_BINS,), jnp.float32)]
            + [pltpu.SemaphoreType.DMA for _ in range(8)]
        ),
    )


def _tc_reduce_body(p_ref, v_ref, o_ref):
    o_ref[...] = jnp.sum(p_ref[...], axis=0, keepdims=True) / v_ref[...]


_tc_reduce = pl.pallas_call(
    _tc_reduce_body,
    out_shape=jax.ShapeDtypeStruct((1, R_BINS * Z_BINS), jnp.float32),
)


def kernel(positions, masses):
    L = N // NSEG
    sc_hist = _make_sc_hist(L, 2048)
    cand = jnp.asarray(_CAND_TAB)
    thr = jnp.asarray(_THR_TAB)
    parts = []
    for q in range(NSEG):
        sl = slice(q * L, (q + 1) * L)
        parts.append(sc_hist(positions[sl, 0], positions[sl, 1],
                             positions[sl, 2], masses[sl], cand, thr))
    partials = jnp.concatenate(parts, axis=0)
    dens = _tc_reduce(partials, jnp.asarray(_VOL_FLAT))
    return dens.reshape(R_BINS, Z_BINS)
